# baseline (speedup 1.0000x reference)
import jax
import jax.numpy as jnp
from jax.experimental import pallas as pl
from jax.experimental.pallas import tpu as pltpu


def _conv3x3_kernel(x_ref, top_ref, bot_ref, w_ref, b_ref, o_ref):
    i = pl.program_id(1)
    last = pl.num_programs(1) - 1
    C = x_ref.shape[1]
    W2 = top_ref.shape[2]
    TH = x_ref.shape[2] // W2

    x = x_ref[0]
    top = jnp.where(i == 0, 0.0, top_ref[0]).astype(x.dtype)
    bot = jnp.where(i == last, 0.0, bot_ref[0]).astype(x.dtype)
    U = jnp.concatenate([top, x, bot], axis=1)

    lane = jax.lax.broadcasted_iota(jnp.int32, (1, U.shape[1]), 1) % W2
    Um = jnp.where(lane == 0, 0.0,
                   jnp.concatenate([U[:, :1], U[:, :-1]], axis=1)).astype(x.dtype)
    Up = jnp.where(lane == W2 - 1, 0.0,
                   jnp.concatenate([U[:, 1:], U[:, -1:]], axis=1)).astype(x.dtype)
    variants = (Um, U, Up)

    acc = pltpu.repeat(b_ref[...], TH, axis=1)
    for ky in range(3):
        lo = ky * W2
        for kx in range(3):
            strip = variants[kx][:, lo:lo + TH * W2]
            acc = acc + jnp.dot(w_ref[ky, kx], strip,
                                preferred_element_type=jnp.float32)
    o_ref[0] = acc.astype(o_ref.dtype)


def kernel(x_nchw, weight, bias):
    N, C, H, W = x_nchw.shape
    H2, W2 = 2 * H, 2 * W
    TH = H2
    for cand in (32, 16, 8, 4, 2, 1):
        if H2 % cand == 0:
            TH = cand
            break

    xu = jnp.repeat(jnp.repeat(x_nchw, 2, axis=2), 2, axis=3).reshape(N, C, H2 * W2)
    wt = jnp.transpose(weight, (2, 3, 0, 1))
    bt = jnp.broadcast_to(bias[:, None], (C, W2)).astype(jnp.float32)

    out = pl.pallas_call(
        _conv3x3_kernel,
        out_shape=jax.ShapeDtypeStruct((N, C, H2 * W2), x_nchw.dtype),
        grid=(N, H2 // TH),
        in_specs=[
            pl.BlockSpec((1, C, TH * W2), lambda n, i: (n, 0, i)),
            pl.BlockSpec((1, C, W2), lambda n, i: (n, 0, jnp.maximum(i * TH - 1, 0))),
            pl.BlockSpec((1, C, W2), lambda n, i: (n, 0, jnp.minimum(i * TH + TH, H2 - 1))),
            pl.BlockSpec((3, 3, C, C), lambda n, i: (0, 0, 0, 0)),
            pl.BlockSpec((C, W2), lambda n, i: (0, 0)),
        ],
        out_specs=pl.BlockSpec((1, C, TH * W2), lambda n, i: (n, 0, i)),
        compiler_params=pltpu.CompilerParams(
            dimension_semantics=("parallel", "parallel"),
            vmem_limit_bytes=64 * 1024 * 1024,
        ),
    )(xu, xu, xu, wt, bt)
    return out.reshape(N, C, H2, W2)


# speedup vs baseline: 2.4237x; 2.4237x over previous
import jax
import jax.numpy as jnp
from jax.experimental import pallas as pl
from jax.experimental.pallas import tpu as pltpu


def _fold_weights_ky(weight_oihw):
    w = weight_oihw
    rows = jnp.stack([
        jnp.stack([w[:, :, 0], w[:, :, 1] + w[:, :, 2]], axis=0),
        jnp.stack([w[:, :, 0] + w[:, :, 1], w[:, :, 2]], axis=0),
    ], axis=0)
    return jnp.moveaxis(rows, -1, 2)


def _expand_mats(W, dtype):
    k = jnp.arange(2 * W)
    j = jnp.arange(4 * W)
    krow, kcol = k // W, k % W
    jrow, jcol = j[None, :] // (2 * W), j[None, :] % (2 * W)
    mats = []
    for kx in range(3):
        dx = kx - 1
        src = jcol + dx
        valid = (src >= 0) & (src < 2 * W)
        hit = (krow[:, None] == jrow) & (kcol[:, None] == src // 2) & valid
        mats.append(hit)
    return jnp.stack(mats, axis=0).astype(dtype)


def _upconv_kernel(x_ref, top_ref, bot_ref, e_ref, w_ref, b_ref, o_ref):
    i = pl.program_id(1)
    last = pl.num_programs(1) - 1
    C = x_ref.shape[1]
    W2 = 2 * (top_ref.shape[2] // 2)
    TH = x_ref.shape[2] // (W2 // 2)

    x = x_ref[0]
    top = jnp.where(i == 0, 0.0, top_ref[0]).astype(x.dtype)
    bot = jnp.where(i == last, 0.0, bot_ref[0]).astype(x.dtype)
    slab = jnp.concatenate([top, x, bot], axis=1)
    n_chunks = slab.shape[1] // W2

    taps = []
    for kx in range(3):
        e = e_ref[kx]
        chunks = [
            jnp.dot(slab[:, c * W2:(c + 1) * W2], e,
                    preferred_element_type=jnp.float32).astype(x.dtype)
            for c in range(n_chunks)
        ]
        taps.append(jnp.concatenate(chunks, axis=1))

    bias = pltpu.repeat(b_ref[...], TH, axis=1)
    for py in range(2):
        acc = bias
        for a in range(2):
            dy = (a - 1) if py == 0 else a
            lo = (2 + dy) * W2
            for kx in range(3):
                strip = taps[kx][:, lo:lo + TH * W2]
                acc = acc + jnp.dot(w_ref[py, a, kx], strip,
                                    preferred_element_type=jnp.float32)
        o_ref[0, :, py::2, :] = acc.reshape(C, TH, W2).astype(o_ref.dtype)


def kernel(x_nchw, weight, bias):
    N, C, H, W = x_nchw.shape
    H2, W2 = 2 * H, 2 * W
    TH = H
    for cand in (16, 8, 4, 2, 1):
        if H % cand == 0:
            TH = cand
            break

    xf = x_nchw.astype(jnp.bfloat16).reshape(N, C, H * W)
    em = _expand_mats(W, jnp.bfloat16)
    wt = _fold_weights_ky(weight).astype(jnp.bfloat16)
    bt = jnp.broadcast_to(bias[:, None], (C, W2)).astype(jnp.float32)

    return pl.pallas_call(
        _upconv_kernel,
        out_shape=jax.ShapeDtypeStruct((N, C, H2, W2), x_nchw.dtype),
        grid=(N, H // TH),
        in_specs=[
            pl.BlockSpec((1, C, TH * W), lambda n, i: (n, 0, i)),
            pl.BlockSpec((1, C, 2 * W),
                         lambda n, i: (n, 0, jnp.maximum(i * (TH // 2) - 1, 0))),
            pl.BlockSpec((1, C, 2 * W),
                         lambda n, i: (n, 0, jnp.minimum((i + 1) * (TH // 2), H // 2 - 1))),
            pl.BlockSpec((3, 2 * W, 4 * W), lambda n, i: (0, 0, 0)),
            pl.BlockSpec((2, 2, 3, C, C), lambda n, i: (0, 0, 0, 0, 0)),
            pl.BlockSpec((C, W2), lambda n, i: (0, 0)),
        ],
        out_specs=pl.BlockSpec((1, C, 2 * TH, W2), lambda n, i: (n, 0, i, 0)),
        compiler_params=pltpu.CompilerParams(
            dimension_semantics=("parallel", "parallel"),
            vmem_limit_bytes=64 * 1024 * 1024,
        ),
    )(xf, xf, xf, em, wt, bt)


# speedup vs baseline: 3.2450x; 1.3389x over previous
import jax
import jax.numpy as jnp
from jax.experimental import pallas as pl
from jax.experimental.pallas import tpu as pltpu


def _fold_weights_ky(weight_oihw):
    w = weight_oihw
    rows = jnp.stack([
        jnp.stack([w[:, :, 0], w[:, :, 1] + w[:, :, 2]], axis=0),
        jnp.stack([w[:, :, 0] + w[:, :, 1], w[:, :, 2]], axis=0),
    ], axis=0)
    return jnp.moveaxis(rows, -1, 2)


def _expand_mats(W, dtype):
    k = jnp.arange(2 * W)
    j = jnp.arange(4 * W)
    krow, kcol = k // W, k % W
    jrow, jcol = j[None, :] // (2 * W), j[None, :] % (2 * W)
    mats = []
    for kx in range(3):
        dx = kx - 1
        src = jcol + dx
        valid = (src >= 0) & (src < 2 * W)
        hit = (krow[:, None] == jrow) & (kcol[:, None] == src // 2) & valid
        mats.append(hit)
    return jnp.stack(mats, axis=0).astype(dtype)


def _upconv_kernel(x_ref, top_ref, bot_ref, e_ref, w_ref, b_ref, o_ref):
    i = pl.program_id(1)
    last = pl.num_programs(1) - 1
    C = x_ref.shape[1]
    W2 = 2 * (top_ref.shape[2] // 2)
    TH = x_ref.shape[2] // (W2 // 2)

    dt = jnp.bfloat16
    x = x_ref[0].astype(dt)
    top = jnp.where(i == 0, 0.0, top_ref[0]).astype(dt)
    bot = jnp.where(i == last, 0.0, bot_ref[0]).astype(dt)
    slab = jnp.concatenate([top, x, bot], axis=1)
    n_chunks = slab.shape[1] // W2

    taps = []
    for kx in range(3):
        e = e_ref[kx]
        chunks = [
            jnp.dot(slab[:, c * W2:(c + 1) * W2], e,
                    preferred_element_type=jnp.float32).astype(dt)
            for c in range(n_chunks)
        ]
        taps.append(jnp.concatenate(chunks, axis=1))
    tap_cat = jnp.concatenate(taps, axis=0)

    bias = pltpu.repeat(b_ref[...], TH, axis=1)
    for py in range(2):
        acc = bias
        for a in range(2):
            dy = (a - 1) if py == 0 else a
            lo = (2 + dy) * W2
            strip = tap_cat[:, lo:lo + TH * W2]
            acc = acc + jnp.dot(w_ref[py, a], strip,
                                preferred_element_type=jnp.float32)
        o_ref[0, :, py::2, :] = acc.reshape(C, TH, W2).astype(o_ref.dtype)


def kernel(x_nchw, weight, bias):
    N, C, H, W = x_nchw.shape
    H2, W2 = 2 * H, 2 * W
    TH = H
    for cand in (32, 16, 8, 4, 2, 1):
        if H % cand == 0:
            TH = cand
            break

    xf = x_nchw.reshape(N, C, H * W)
    em = _expand_mats(W, jnp.bfloat16)
    wt = _fold_weights_ky(weight).astype(jnp.bfloat16)
    wt = jnp.transpose(wt, (0, 1, 3, 2, 4)).reshape(2, 2, C, 3 * C)
    bt = jnp.broadcast_to(bias[:, None], (C, W2)).astype(jnp.float32)

    return pl.pallas_call(
        _upconv_kernel,
        out_shape=jax.ShapeDtypeStruct((N, C, H2, W2), x_nchw.dtype),
        grid=(N, H // TH),
        in_specs=[
            pl.BlockSpec((1, C, TH * W), lambda n, i: (n, 0, i)),
            pl.BlockSpec((1, C, 2 * W),
                         lambda n, i: (n, 0, jnp.maximum(i * (TH // 2) - 1, 0))),
            pl.BlockSpec((1, C, 2 * W),
                         lambda n, i: (n, 0, jnp.minimum((i + 1) * (TH // 2), H // 2 - 1))),
            pl.BlockSpec((3, 2 * W, 4 * W), lambda n, i: (0, 0, 0)),
            pl.BlockSpec((2, 2, C, 3 * C), lambda n, i: (0, 0, 0, 0)),
            pl.BlockSpec((C, W2), lambda n, i: (0, 0)),
        ],
        out_specs=pl.BlockSpec((1, C, 2 * TH, W2), lambda n, i: (n, 0, i, 0)),
        compiler_params=pltpu.CompilerParams(
            dimension_semantics=("parallel", "parallel"),
            vmem_limit_bytes=64 * 1024 * 1024,
        ),
    )(xf, xf, xf, em, wt, bt)
